# baseline (speedup 1.0000x reference)
import functools

import jax
import jax.numpy as jnp
from jax import lax
from jax.experimental import pallas as pl
from jax.experimental.pallas import tpu as pltpu
from jax.experimental.pallas import tpu_sc as plsc

N = 10000
NP = 10240
E = 320000
NC = 2
NS = 16
EPS = E // NS

_MESH = plsc.VectorSubcoreMesh(core_axis_name="c", subcore_axis_name="s")

ROWS_PER_SUB = NP // NS
C2 = 40
NBUF = 4
NCH = EPS // C2
NG = NCH // NBUF
NCH0 = E // (NC * NS) // C2
NG0 = NCH0 // NBUF


def _sc_agg_generic(tables, ei_hbm, dst_hbm, out_hbm, cnt_hbm, zeros_hbm,
                    ones_hbm, scr, layer0):
    srcv = (scr[0:NBUF], scr[NBUF:2 * NBUF])
    dstv = (scr[2 * NBUF:3 * NBUF], scr[3 * NBUF:4 * NBUF])
    rows = (scr[4 * NBUF:5 * NBUF], scr[5 * NBUF:6 * NBUF])
    k = 6 * NBUF
    if layer0:
        onesv = scr[k]
        k += 1
    acc = scr[k]
    isem = scr[k + 1:k + 1 + NBUF]
    gsem = scr[k + 1 + NBUF:k + 1 + 2 * NBUF]
    ssem = scr[k + 1 + 2 * NBUF:k + 1 + 3 * NBUF]
    c = lax.axis_index("c")
    s = lax.axis_index("s")
    rbase = s * ROWS_PER_SUB
    rslice = pl.ds(rbase, ROWS_PER_SUB)
    ooff = c * NP
    if layer0:
        ebase = (c * NS + s) * (E // (NC * NS))
        nch, ng = NCH0, NG0
    else:
        ebase = s * EPS
        nch, ng = NCH, NG

    def fetch_idx(g, p, b, sem):
        off = ebase + (g * NBUF + b) * C2
        pltpu.async_copy(ei_hbm.at[pl.ds(off, C2)], srcv[p][b], sem)
        pltpu.async_copy(dst_hbm.at[pl.ds(off, C2)], dstv[p][b], sem)

    def wait_idx(p, b, sem):
        pltpu.make_async_copy(dst_hbm.at[pl.ds(0, C2)], srcv[p][b], sem).wait()
        pltpu.make_async_copy(dst_hbm.at[pl.ds(0, C2)], dstv[p][b], sem).wait()

    def issue_gather(p, b, sem):
        if layer0:
            pltpu.async_copy(tables[0].at[srcv[p][b]], rows[p][b], sem)
        else:
            @pl.when(c == 0)
            def _():
                pltpu.async_copy(tables[0].at[srcv[p][b]], rows[p][b], sem)

            @pl.when(c == 1)
            def _():
                pltpu.async_copy(tables[1].at[srcv[p][b]], rows[p][b], sem)

    def wait_gather(p, b, sem):
        pltpu.make_async_copy(tables[0].at[srcv[p][b]], rows[p][b], sem).wait()

    pltpu.sync_copy(zeros_hbm.at[rslice], acc.at[rslice])
    if layer0:
        pltpu.sync_copy(ones_hbm, onesv)
    plsc.subcore_barrier()

    if layer0:
        for b in range(NBUF):
            pltpu.async_copy(dst_hbm.at[pl.ds(ebase + b * C2, C2)],
                             dstv[0][b], isem[b])

        def cbody(g, p, q, first):
            for b in range(NBUF):
                pltpu.make_async_copy(dst_hbm.at[pl.ds(0, C2)],
                                      dstv[p][b], isem[b]).wait()
                pltpu.async_copy(onesv, acc.at[dstv[p][b]], ssem[b], add=True)
            for b in range(NBUF):
                if not first:
                    pltpu.make_async_copy(onesv, acc.at[dstv[q][b]], ssem[b]).wait()

                @pl.when(g + 1 < NG0)
                def _():
                    off = ebase + ((g + 1) * NBUF + b) * C2
                    pltpu.async_copy(dst_hbm.at[pl.ds(off, C2)],
                                     dstv[q][b], isem[b])

        cbody(0, 0, 1, True)

        @pl.loop(1, NG0)
        def _(g):
            @pl.when(g % 2 == 1)
            def _():
                cbody(g, 1, 0, False)

            @pl.when(g % 2 == 0)
            def _():
                cbody(g, 0, 1, False)

        pl_last = (NG0 - 1) % 2
        for b in range(NBUF):
            pltpu.make_async_copy(onesv, acc.at[dstv[pl_last][b]], ssem[b]).wait()
        for t in range(NG0 * NBUF, NCH0):
            pltpu.sync_copy(dst_hbm.at[pl.ds(ebase + t * C2, C2)], dstv[0][0])
            pltpu.sync_copy(onesv, acc.at[dstv[0][0]], add=True)

        plsc.subcore_barrier()
        pltpu.sync_copy(acc.at[rslice], cnt_hbm.at[pl.ds(ooff + rbase, ROWS_PER_SUB)])
        plsc.subcore_barrier()
        pltpu.sync_copy(zeros_hbm.at[rslice], acc.at[rslice])
        plsc.subcore_barrier()

    for b in range(NBUF):
        fetch_idx(0, 0, b, isem[b])

    def gbody(g, p, q, first):
        for b in range(NBUF):
            wait_idx(p, b, isem[b])
            issue_gather(p, b, gsem[b])
        for b in range(NBUF):
            if not first:
                pltpu.make_async_copy(rows[q][b], acc.at[dstv[q][b]], ssem[b]).wait()

            @pl.when(g + 1 < ng)
            def _():
                fetch_idx(g + 1, q, b, isem[b])
        for b in range(NBUF):
            wait_gather(p, b, gsem[b])
            pltpu.async_copy(rows[p][b], acc.at[dstv[p][b]], ssem[b], add=True)

    gbody(0, 0, 1, True)

    @pl.loop(1, ng)
    def _(g):
        @pl.when(g % 2 == 1)
        def _():
            gbody(g, 1, 0, False)

        @pl.when(g % 2 == 0)
        def _():
            gbody(g, 0, 1, False)

    pl_last = (ng - 1) % 2
    for b in range(NBUF):
        pltpu.make_async_copy(rows[pl_last][b], acc.at[dstv[pl_last][b]], ssem[b]).wait()
    for t in range(ng * NBUF, nch):
        pltpu.sync_copy(ei_hbm.at[pl.ds(ebase + t * C2, C2)], srcv[0][0])
        pltpu.sync_copy(dst_hbm.at[pl.ds(ebase + t * C2, C2)], dstv[0][0])
        issue_gather(0, 0, gsem[0])
        wait_gather(0, 0, gsem[0])
        pltpu.sync_copy(rows[0][0], acc.at[dstv[0][0]], add=True)

    plsc.subcore_barrier()
    pltpu.sync_copy(acc.at[rslice], out_hbm.at[pl.ds(ooff + rbase, ROWS_PER_SUB)])


def _sc_agg0_body(table, ei_hbm, dst_hbm, zeros_hbm, ones_hbm, out_hbm,
                  cnt_hbm, *scr):
    _sc_agg_generic((table,), ei_hbm, dst_hbm, out_hbm, cnt_hbm, zeros_hbm,
                    ones_hbm, scr, layer0=True)


def _sc_agg_body(t0, t1, ei_hbm, dst_hbm, zeros_hbm, out_hbm, *scr):
    _sc_agg_generic((t0, t1), ei_hbm, dst_hbm, out_hbm, None, zeros_hbm,
                    None, scr, layer0=False)


def _ring_scratch(with_ones):
    scr = [pltpu.VMEM((C2,), jnp.int32) for _ in range(4 * NBUF)]
    scr += [pltpu.VMEM((C2, 128), jnp.float32) for _ in range(2 * NBUF)]
    if with_ones:
        scr.append(pltpu.VMEM((C2, 128), jnp.float32))
    scr.append(pltpu.VMEM_SHARED((NP, 128), jnp.float32))
    scr += [pltpu.SemaphoreType.DMA for _ in range(3 * NBUF)]
    return scr


_sc_agg0 = pl.kernel(
    _sc_agg0_body,
    out_type=(jax.ShapeDtypeStruct((2 * NP, 128), jnp.float32),
              jax.ShapeDtypeStruct((2 * NP, 128), jnp.float32)),
    mesh=_MESH,
    scratch_types=_ring_scratch(True),
)

_sc_agg = pl.kernel(
    _sc_agg_body,
    out_type=(jax.ShapeDtypeStruct((2 * NP, 128), jnp.float32),),
    mesh=_MESH,
    scratch_types=_ring_scratch(False),
)

_R = 2000
_GRID = N // _R


def _self_body(h_ref, w_ref, b_ref, o_ref):
    o_ref[...] = (
        jnp.dot(h_ref[...], w_ref[...], preferred_element_type=jnp.float32)
        + b_ref[...]
    )


def _tc_self(h, wT, b):
    din = h.shape[1]
    return pl.pallas_call(
        _self_body,
        grid=(_GRID,),
        in_specs=[
            pl.BlockSpec((_R, din), lambda i: (i, 0)),
            pl.BlockSpec((din, 256), lambda i: (0, 0)),
            pl.BlockSpec((1, 256), lambda i: (0, 0)),
        ],
        out_specs=pl.BlockSpec((_R, 256), lambda i: (i, 0)),
        out_shape=jax.ShapeDtypeStruct((N, 256), jnp.float32),
    )(h, wT, b.reshape(1, 256))


def _self_split_body(h0_ref, h1_ref, w_ref, b_ref, o_ref):
    h = jnp.concatenate([h0_ref[...], h1_ref[...]], axis=1)
    o_ref[...] = (
        jnp.dot(h, w_ref[...], preferred_element_type=jnp.float32) + b_ref[...]
    )


def _tc_self_split(h0, h1, wT, b):
    return pl.pallas_call(
        _self_split_body,
        grid=(_GRID,),
        in_specs=[
            pl.BlockSpec((_R, 128), lambda i: (i, 0)),
            pl.BlockSpec((_R, 128), lambda i: (i, 0)),
            pl.BlockSpec((256, 256), lambda i: (0, 0)),
            pl.BlockSpec((1, 256), lambda i: (0, 0)),
        ],
        out_specs=pl.BlockSpec((_R, 256), lambda i: (i, 0)),
        out_shape=jax.ShapeDtypeStruct((N, 256), jnp.float32),
    )(h0, h1, wT, b.reshape(1, 256))


def _comb_body(relu, concat, rc_in, split_out, heads, agg_ref, cd_ref,
               selfp_ref, wr_ref, *refs):
    if heads:
        wh_ref, bh_ref = refs[0], refs[1]
        out_refs = refs[2:]
    else:
        out_refs = refs
    if rc_in:
        rc = cd_ref[:, :1]
    else:
        cnt = cd_ref[0, :, :1] + cd_ref[1, :, :1]
        rc = 1.0 / jnp.maximum(cnt, 1.0)
    if concat:
        agg = jnp.concatenate([agg_ref[0], agg_ref[1]], axis=1)
    else:
        agg = agg_ref[0] + agg_ref[1]
    acc = selfp_ref[...] + jnp.dot(agg * rc, wr_ref[...],
                                   preferred_element_type=jnp.float32)
    h = jnp.maximum(acc, 0.0) if relu else acc
    if split_out:
        out_refs[0][...] = h[:, :128]
        out_refs[1][...] = h[:, 128:]
        i = 2
    else:
        out_refs[0][...] = h
        i = 1
    if not rc_in:
        out_refs[i][...] = jnp.broadcast_to(rc, (rc.shape[0], 8))
        i += 1
    if heads:
        res = (jnp.dot(h, wh_ref[...], preferred_element_type=jnp.float32)
               + bh_ref[...])
        out_refs[i][...] = res[:, :50]
        out_refs[i + 1][...] = res[:, 50:178]
        out_refs[i + 2][...] = res[:, 178:210]


def _tc_combine(agg, cd, selfp, wrT, relu, concat, split_out, heads=None):
    dh = agg.shape[2]
    rc_in = cd.ndim == 2
    if rc_in:
        cd_spec = pl.BlockSpec((_R, 8), lambda i: (i, 0))
    else:
        cd_spec = pl.BlockSpec((2, _R, 128), lambda i: (0, i, 0))
    in_specs = [
        pl.BlockSpec((2, _R, dh), lambda i: (0, i, 0)),
        cd_spec,
        pl.BlockSpec((_R, 256), lambda i: (i, 0)),
        pl.BlockSpec((wrT.shape[0], 256), lambda i: (0, 0)),
    ]
    args = [agg, cd, selfp, wrT]
    out_specs, out_shape = [], []
    if split_out:
        for _ in range(2):
            out_specs.append(pl.BlockSpec((_R, 128), lambda i: (i, 0)))
            out_shape.append(jax.ShapeDtypeStruct((N, 128), jnp.float32))
    else:
        out_specs.append(pl.BlockSpec((_R, 256), lambda i: (i, 0)))
        out_shape.append(jax.ShapeDtypeStruct((N, 256), jnp.float32))
    if not rc_in:
        out_specs.append(pl.BlockSpec((_R, 8), lambda i: (i, 0)))
        out_shape.append(jax.ShapeDtypeStruct((N, 8), jnp.float32))
    if heads is not None:
        wh, bh = heads
        in_specs.append(pl.BlockSpec((256, 256), lambda i: (0, 0)))
        in_specs.append(pl.BlockSpec((1, 256), lambda i: (0, 0)))
        args.extend([wh, bh.reshape(1, 256)])
        for w in (50, 128, 32):
            out_specs.append(pl.BlockSpec((_R, w), lambda i: (i, 0)))
            out_shape.append(jax.ShapeDtypeStruct((N, w), jnp.float32))
    return pl.pallas_call(
        functools.partial(_comb_body, relu, concat, rc_in, split_out,
                          heads is not None),
        grid=(_GRID,),
        in_specs=in_specs,
        out_specs=out_specs,
        out_shape=out_shape,
    )(*args)


def kernel(x, edge_index, Wl0, Wr0, b0, Wl1, Wr1, b1, Wl2, Wr2, b2,
           We, be, Wq, bq, Wd, bd):
    src_ids = edge_index[0]
    dst = edge_index[1]
    z128 = jnp.zeros((NP, 128), jnp.float32)
    ones = jnp.ones((C2, 128), jnp.float32)

    agg0, cnt = _sc_agg0(x, src_ids, dst, z128, ones)
    self0 = _tc_self(x, Wl0.T, b0)
    h0a, h0b, rc = _tc_combine(agg0.reshape(2, NP, 128), cnt.reshape(2, NP, 128),
                               self0, Wr0.T, True, False, True)
    (agg1,) = _sc_agg(h0a, h0b, src_ids, dst, z128)
    self1 = _tc_self_split(h0a, h0b, Wl1.T, b1)
    h1a, h1b = _tc_combine(agg1.reshape(2, NP, 128), rc, self1, Wr1.T,
                           True, True, True)
    (agg2,) = _sc_agg(h1a, h1b, src_ids, dst, z128)
    self2 = _tc_self_split(h1a, h1b, Wl2.T, b2)
    wcat = jnp.concatenate([We, Wq, Wd], axis=0).T
    wcat = jnp.pad(wcat, ((0, 0), (0, 46)))
    bcat = jnp.pad(jnp.concatenate([be, bq, bd]), (0, 46))
    h, entity, query, domain = _tc_combine(agg2.reshape(2, NP, 128), rc,
                                           self2, Wr2.T, False, True, False,
                                           heads=(wcat, bcat))
    return (h, entity, query, domain)


# speedup vs baseline: 1.0147x; 1.0147x over previous
import functools

import jax
import jax.numpy as jnp
from jax import lax
from jax.experimental import pallas as pl
from jax.experimental.pallas import tpu as pltpu
from jax.experimental.pallas import tpu_sc as plsc

N = 10000
NP = 10240
E = 320000
NC = 2
NS = 16
EPS = E // NS

_MESH = plsc.VectorSubcoreMesh(core_axis_name="c", subcore_axis_name="s")

ROWS_PER_SUB = NP // NS
C2 = 40
NBUF = 4
NCH = EPS // C2
NG = NCH // NBUF
NCH0 = E // (NC * NS) // C2
NG0 = NCH0 // NBUF


def _sc_agg_generic(tables, src_hbm, out_hbm, cnt_hbm, zeros_hbm,
                    ones_hbm, scr, layer0):
    srcv = (scr[0:NBUF], scr[NBUF:2 * NBUF])
    dstv = (scr[2 * NBUF:3 * NBUF], scr[3 * NBUF:4 * NBUF])
    rows = (scr[4 * NBUF:5 * NBUF], scr[5 * NBUF:6 * NBUF])
    k = 6 * NBUF
    if layer0:
        onesv = scr[k]
        k += 1
    acc = scr[k]
    isem = scr[k + 1:k + 1 + NBUF]
    gsem = scr[k + 1 + NBUF:k + 1 + 2 * NBUF]
    ssem = scr[k + 1 + 2 * NBUF:k + 1 + 3 * NBUF]
    c = lax.axis_index("c")
    s = lax.axis_index("s")
    rbase = s * ROWS_PER_SUB
    rslice = pl.ds(rbase, ROWS_PER_SUB)
    ooff = c * NP
    if layer0:
        ebase = (c * NS + s) * (E // (NC * NS))
        nch, ng = NCH0, NG0
    else:
        ebase = s * EPS
        nch, ng = NCH, NG

    def fetch_idx(g, p, b, sem):
        off = ebase + (g * NBUF + b) * C2
        pltpu.async_copy(src_hbm.at[pl.ds(off, C2)], srcv[p][b], sem)
        pltpu.async_copy(src_hbm.at[pl.ds(E + off, C2)], dstv[p][b], sem)

    def wait_idx(p, b, sem):
        pltpu.make_async_copy(src_hbm.at[pl.ds(0, C2)], srcv[p][b], sem).wait()
        pltpu.make_async_copy(src_hbm.at[pl.ds(0, C2)], dstv[p][b], sem).wait()

    def issue_gather(p, b, sem):
        if layer0:
            pltpu.async_copy(tables[0].at[srcv[p][b]], rows[p][b], sem)
        else:
            @pl.when(c == 0)
            def _():
                pltpu.async_copy(tables[0].at[srcv[p][b]], rows[p][b], sem)

            @pl.when(c == 1)
            def _():
                pltpu.async_copy(tables[1].at[srcv[p][b]], rows[p][b], sem)

    def wait_gather(p, b, sem):
        pltpu.make_async_copy(tables[0].at[srcv[p][b]], rows[p][b], sem).wait()

    pltpu.sync_copy(zeros_hbm.at[rslice], acc.at[rslice])
    if layer0:
        pltpu.sync_copy(ones_hbm, onesv)
    plsc.subcore_barrier()

    if layer0:
        for b in range(NBUF):
            pltpu.async_copy(src_hbm.at[pl.ds(E + ebase + b * C2, C2)],
                             dstv[0][b], isem[b])

        def cbody(g, p, q, first):
            for b in range(NBUF):
                pltpu.make_async_copy(src_hbm.at[pl.ds(0, C2)],
                                      dstv[p][b], isem[b]).wait()
                pltpu.async_copy(onesv, acc.at[dstv[p][b]], ssem[b], add=True)
            for b in range(NBUF):
                if not first:
                    pltpu.make_async_copy(onesv, acc.at[dstv[q][b]], ssem[b]).wait()

                @pl.when(g + 1 < NG0)
                def _():
                    off = E + ebase + ((g + 1) * NBUF + b) * C2
                    pltpu.async_copy(src_hbm.at[pl.ds(off, C2)],
                                     dstv[q][b], isem[b])

        cbody(0, 0, 1, True)

        @pl.loop(1, NG0)
        def _(g):
            @pl.when(g % 2 == 1)
            def _():
                cbody(g, 1, 0, False)

            @pl.when(g % 2 == 0)
            def _():
                cbody(g, 0, 1, False)

        pl_last = (NG0 - 1) % 2
        for b in range(NBUF):
            pltpu.make_async_copy(onesv, acc.at[dstv[pl_last][b]], ssem[b]).wait()
        for t in range(NG0 * NBUF, NCH0):
            pltpu.sync_copy(src_hbm.at[pl.ds(E + ebase + t * C2, C2)], dstv[0][0])
            pltpu.sync_copy(onesv, acc.at[dstv[0][0]], add=True)

        plsc.subcore_barrier()
        pltpu.sync_copy(acc.at[rslice], cnt_hbm.at[pl.ds(ooff + rbase, ROWS_PER_SUB)])
        plsc.subcore_barrier()
        pltpu.sync_copy(zeros_hbm.at[rslice], acc.at[rslice])
        plsc.subcore_barrier()

    for b in range(NBUF):
        fetch_idx(0, 0, b, isem[b])

    def gbody(g, p, q, first):
        for b in range(NBUF):
            wait_idx(p, b, isem[b])
            issue_gather(p, b, gsem[b])
        for b in range(NBUF):
            if not first:
                pltpu.make_async_copy(rows[q][b], acc.at[dstv[q][b]], ssem[b]).wait()

            @pl.when(g + 1 < ng)
            def _():
                fetch_idx(g + 1, q, b, isem[b])
        for b in range(NBUF):
            wait_gather(p, b, gsem[b])
            pltpu.async_copy(rows[p][b], acc.at[dstv[p][b]], ssem[b], add=True)

    gbody(0, 0, 1, True)

    @pl.loop(1, ng)
    def _(g):
        @pl.when(g % 2 == 1)
        def _():
            gbody(g, 1, 0, False)

        @pl.when(g % 2 == 0)
        def _():
            gbody(g, 0, 1, False)

    pl_last = (ng - 1) % 2
    for b in range(NBUF):
        pltpu.make_async_copy(rows[pl_last][b], acc.at[dstv[pl_last][b]], ssem[b]).wait()
    for t in range(ng * NBUF, nch):
        pltpu.sync_copy(src_hbm.at[pl.ds(ebase + t * C2, C2)], srcv[0][0])
        pltpu.sync_copy(src_hbm.at[pl.ds(E + ebase + t * C2, C2)], dstv[0][0])
        issue_gather(0, 0, gsem[0])
        wait_gather(0, 0, gsem[0])
        pltpu.sync_copy(rows[0][0], acc.at[dstv[0][0]], add=True)

    plsc.subcore_barrier()
    pltpu.sync_copy(acc.at[rslice], out_hbm.at[pl.ds(ooff + rbase, ROWS_PER_SUB)])


def _sc_agg0_body(table, src_hbm, zeros_hbm, ones_hbm, out_hbm, cnt_hbm, *scr):
    _sc_agg_generic((table,), src_hbm, out_hbm, cnt_hbm, zeros_hbm,
                    ones_hbm, scr, layer0=True)


def _sc_agg_body(t0, t1, src_hbm, zeros_hbm, out_hbm, *scr):
    _sc_agg_generic((t0, t1), src_hbm, out_hbm, None, zeros_hbm,
                    None, scr, layer0=False)


def _ring_scratch(with_ones):
    scr = [pltpu.VMEM((C2,), jnp.int32) for _ in range(4 * NBUF)]
    scr += [pltpu.VMEM((C2, 128), jnp.float32) for _ in range(2 * NBUF)]
    if with_ones:
        scr.append(pltpu.VMEM((C2, 128), jnp.float32))
    scr.append(pltpu.VMEM_SHARED((NP, 128), jnp.float32))
    scr += [pltpu.SemaphoreType.DMA for _ in range(3 * NBUF)]
    return scr


_sc_agg0 = pl.kernel(
    _sc_agg0_body,
    out_type=(jax.ShapeDtypeStruct((2 * NP, 128), jnp.float32),
              jax.ShapeDtypeStruct((2 * NP, 128), jnp.float32)),
    mesh=_MESH,
    scratch_types=_ring_scratch(True),
)

_sc_agg = pl.kernel(
    _sc_agg_body,
    out_type=(jax.ShapeDtypeStruct((2 * NP, 128), jnp.float32),),
    mesh=_MESH,
    scratch_types=_ring_scratch(False),
)

_R = 2000
_GRID = N // _R


def _self_body(h_ref, w_ref, b_ref, o_ref):
    o_ref[...] = (
        jnp.dot(h_ref[...], w_ref[...], preferred_element_type=jnp.float32)
        + b_ref[...]
    )


def _tc_self(h, wT, b):
    din = h.shape[1]
    return pl.pallas_call(
        _self_body,
        grid=(_GRID,),
        in_specs=[
            pl.BlockSpec((_R, din), lambda i: (i, 0)),
            pl.BlockSpec((din, 256), lambda i: (0, 0)),
            pl.BlockSpec((1, 256), lambda i: (0, 0)),
        ],
        out_specs=pl.BlockSpec((_R, 256), lambda i: (i, 0)),
        out_shape=jax.ShapeDtypeStruct((N, 256), jnp.float32),
    )(h, wT, b.reshape(1, 256))


def _self_split_body(h0_ref, h1_ref, w_ref, b_ref, o_ref):
    h = jnp.concatenate([h0_ref[...], h1_ref[...]], axis=1)
    o_ref[...] = (
        jnp.dot(h, w_ref[...], preferred_element_type=jnp.float32) + b_ref[...]
    )


def _tc_self_split(h0, h1, wT, b):
    return pl.pallas_call(
        _self_split_body,
        grid=(_GRID,),
        in_specs=[
            pl.BlockSpec((_R, 128), lambda i: (i, 0)),
            pl.BlockSpec((_R, 128), lambda i: (i, 0)),
            pl.BlockSpec((256, 256), lambda i: (0, 0)),
            pl.BlockSpec((1, 256), lambda i: (0, 0)),
        ],
        out_specs=pl.BlockSpec((_R, 256), lambda i: (i, 0)),
        out_shape=jax.ShapeDtypeStruct((N, 256), jnp.float32),
    )(h0, h1, wT, b.reshape(1, 256))


def _comb_body(relu, concat, rc_in, split_out, heads, agg_ref, cd_ref,
               selfp_ref, wr_ref, *refs):
    if heads:
        wh_ref, bh_ref = refs[0], refs[1]
        out_refs = refs[2:]
    else:
        out_refs = refs
    if rc_in:
        rc = cd_ref[:, :1]
    else:
        cnt = cd_ref[0, :, :1] + cd_ref[1, :, :1]
        rc = 1.0 / jnp.maximum(cnt, 1.0)
    if concat:
        agg = jnp.concatenate([agg_ref[0], agg_ref[1]], axis=1)
    else:
        agg = agg_ref[0] + agg_ref[1]
    acc = selfp_ref[...] + jnp.dot(agg * rc, wr_ref[...],
                                   preferred_element_type=jnp.float32)
    h = jnp.maximum(acc, 0.0) if relu else acc
    if split_out:
        out_refs[0][...] = h[:, :128]
        out_refs[1][...] = h[:, 128:]
        i = 2
    else:
        out_refs[0][...] = h
        i = 1
    if not rc_in:
        out_refs[i][...] = jnp.broadcast_to(rc, (rc.shape[0], 8))
        i += 1
    if heads:
        res = (jnp.dot(h, wh_ref[...], preferred_element_type=jnp.float32)
               + bh_ref[...])
        out_refs[i][...] = res[:, :50]
        out_refs[i + 1][...] = res[:, 50:178]
        out_refs[i + 2][...] = res[:, 178:210]


def _tc_combine(agg, cd, selfp, wrT, relu, concat, split_out, heads=None):
    dh = agg.shape[2]
    rc_in = cd.ndim == 2
    if rc_in:
        cd_spec = pl.BlockSpec((_R, 8), lambda i: (i, 0))
    else:
        cd_spec = pl.BlockSpec((2, _R, 128), lambda i: (0, i, 0))
    in_specs = [
        pl.BlockSpec((2, _R, dh), lambda i: (0, i, 0)),
        cd_spec,
        pl.BlockSpec((_R, 256), lambda i: (i, 0)),
        pl.BlockSpec((wrT.shape[0], 256), lambda i: (0, 0)),
    ]
    args = [agg, cd, selfp, wrT]
    out_specs, out_shape = [], []
    if split_out:
        for _ in range(2):
            out_specs.append(pl.BlockSpec((_R, 128), lambda i: (i, 0)))
            out_shape.append(jax.ShapeDtypeStruct((N, 128), jnp.float32))
    else:
        out_specs.append(pl.BlockSpec((_R, 256), lambda i: (i, 0)))
        out_shape.append(jax.ShapeDtypeStruct((N, 256), jnp.float32))
    if not rc_in:
        out_specs.append(pl.BlockSpec((_R, 8), lambda i: (i, 0)))
        out_shape.append(jax.ShapeDtypeStruct((N, 8), jnp.float32))
    if heads is not None:
        wh, bh = heads
        in_specs.append(pl.BlockSpec((256, 256), lambda i: (0, 0)))
        in_specs.append(pl.BlockSpec((1, 256), lambda i: (0, 0)))
        args.extend([wh, bh.reshape(1, 256)])
        for w in (50, 128, 32):
            out_specs.append(pl.BlockSpec((_R, w), lambda i: (i, 0)))
            out_shape.append(jax.ShapeDtypeStruct((N, w), jnp.float32))
    return pl.pallas_call(
        functools.partial(_comb_body, relu, concat, rc_in, split_out,
                          heads is not None),
        grid=(_GRID,),
        in_specs=in_specs,
        out_specs=out_specs,
        out_shape=out_shape,
    )(*args)


def kernel(x, edge_index, Wl0, Wr0, b0, Wl1, Wr1, b1, Wl2, Wr2, b2,
           We, be, Wq, bq, Wd, bd):
    eflat = edge_index.reshape(2 * E)
    z128 = jnp.zeros((NP, 128), jnp.float32)
    ones = jnp.ones((C2, 128), jnp.float32)

    agg0, cnt = _sc_agg0(x, eflat, z128, ones)
    self0 = _tc_self(x, Wl0.T, b0)
    h0a, h0b, rc = _tc_combine(agg0.reshape(2, NP, 128), cnt.reshape(2, NP, 128),
                               self0, Wr0.T, True, False, True)
    (agg1,) = _sc_agg(h0a, h0b, eflat, z128)
    self1 = _tc_self_split(h0a, h0b, Wl1.T, b1)
    h1a, h1b = _tc_combine(agg1.reshape(2, NP, 128), rc, self1, Wr1.T,
                           True, True, True)
    (agg2,) = _sc_agg(h1a, h1b, eflat, z128)
    self2 = _tc_self_split(h1a, h1b, Wl2.T, b2)
    wcat = jnp.concatenate([We, Wq, Wd], axis=0).T
    wcat = jnp.pad(wcat, ((0, 0), (0, 46)))
    bcat = jnp.pad(jnp.concatenate([be, bq, bd]), (0, 46))
    h, entity, query, domain = _tc_combine(agg2.reshape(2, NP, 128), rc,
                                           self2, Wr2.T, False, True, False,
                                           heads=(wcat, bcat))
    return (h, entity, query, domain)
